# baseline (device time: 183822 ns/iter reference)
import jax
import jax.numpy as jnp
from jax import lax
from jax.experimental import pallas as pl
from jax.experimental.pallas import tpu as pltpu

N_DEV = 8


def kernel(x, w_mat):
    m_per, k = x.shape
    _, n_per = w_mat.shape

    def body(x_ref, w_ref, out_ref, comm_ref, send_sems, recv_sems):
        my = lax.axis_index("i")
        left = lax.rem(my + (N_DEV - 1), N_DEV)
        right = lax.rem(my + 1, N_DEV)

        barrier_sem = pltpu.get_barrier_semaphore()
        pl.semaphore_signal(
            barrier_sem, inc=1, device_id=(left,),
            device_id_type=pl.DeviceIdType.MESH,
        )
        pl.semaphore_signal(
            barrier_sem, inc=1, device_id=(right,),
            device_id_type=pl.DeviceIdType.MESH,
        )
        pl.semaphore_wait(barrier_sem, 2)

        out_ref[pl.ds(my * m_per, m_per), :] = jnp.dot(
            x_ref[...], w_ref[...], preferred_element_type=jnp.float32
        )

        for h in range(N_DEV - 1):
            src = x_ref if h == 0 else comm_ref.at[h - 1]
            rdma = pltpu.make_async_remote_copy(
                src_ref=src,
                dst_ref=comm_ref.at[h],
                send_sem=send_sems.at[h],
                recv_sem=recv_sems.at[h],
                device_id=(right,),
                device_id_type=pl.DeviceIdType.MESH,
            )
            rdma.start()
            rdma.wait()
            origin = lax.rem(my + (N_DEV - 1 - h), N_DEV)
            out_ref[pl.ds(origin * m_per, m_per), :] = jnp.dot(
                comm_ref[h], w_ref[...], preferred_element_type=jnp.float32
            )

    return pl.pallas_call(
        body,
        out_shape=jax.ShapeDtypeStruct((N_DEV * m_per, n_per), jnp.float32),
        in_specs=[
            pl.BlockSpec(memory_space=pltpu.VMEM),
            pl.BlockSpec(memory_space=pltpu.VMEM),
        ],
        out_specs=pl.BlockSpec(memory_space=pltpu.VMEM),
        scratch_shapes=[
            pltpu.VMEM((N_DEV - 1, m_per, k), jnp.float32),
            pltpu.SemaphoreType.DMA((N_DEV - 1,)),
            pltpu.SemaphoreType.DMA((N_DEV - 1,)),
        ],
        compiler_params=pltpu.CompilerParams(collective_id=0),
    )(x, w_mat)


# device time: 94779 ns/iter; 1.9395x vs baseline; 1.9395x over previous
import jax
import jax.numpy as jnp
from jax import lax
from jax.experimental import pallas as pl
from jax.experimental.pallas import tpu as pltpu

N_DEV = 8
N_HOP = 4


def kernel(x, w_mat):
    m_per, k = x.shape
    _, n_per = w_mat.shape
    half = m_per // 2

    def body(x_ref, w_ref, out_ref, comm_ref,
             cw_send, cw_recv, ccw_send, ccw_recv):
        my = lax.axis_index("i")
        left = lax.rem(my + (N_DEV - 1), N_DEV)
        right = lax.rem(my + 1, N_DEV)

        barrier_sem = pltpu.get_barrier_semaphore()
        for nbr in (left, right):
            pl.semaphore_signal(
                barrier_sem, inc=1, device_id=(nbr,),
                device_id_type=pl.DeviceIdType.MESH,
            )
        pl.semaphore_wait(barrier_sem, 2)

        def make(src, dst, sends, recvs, h, dev):
            return pltpu.make_async_remote_copy(
                src_ref=src, dst_ref=dst,
                send_sem=sends.at[h], recv_sem=recvs.at[h],
                device_id=(dev,), device_id_type=pl.DeviceIdType.MESH,
            )

        cw = [
            make(x_ref, comm_ref.at[0], cw_send, cw_recv, 0, right),
            make(comm_ref.at[0], comm_ref.at[1], cw_send, cw_recv, 1, right),
            make(comm_ref.at[1], comm_ref.at[2], cw_send, cw_recv, 2, right),
            make(comm_ref.at[2, pl.ds(0, half)], comm_ref.at[3, pl.ds(0, half)],
                 cw_send, cw_recv, 3, right),
        ]
        ccw = [
            make(x_ref, comm_ref.at[6], ccw_send, ccw_recv, 0, left),
            make(comm_ref.at[6], comm_ref.at[5], ccw_send, ccw_recv, 1, left),
            make(comm_ref.at[5], comm_ref.at[4], ccw_send, ccw_recv, 2, left),
            make(comm_ref.at[4, pl.ds(half, half)],
                 comm_ref.at[3, pl.ds(half, half)],
                 ccw_send, ccw_recv, 3, left),
        ]

        def gemm(r):
            origin = lax.rem(my + (N_DEV - r), N_DEV)
            src = x_ref if r == 0 else comm_ref.at[r - 1]
            out_ref[pl.ds(origin * m_per, m_per), :] = jnp.dot(
                src[...], w_ref[...], preferred_element_type=jnp.float32
            )

        cw[0].start()
        ccw[0].start()
        gemm(0)
        for h in range(N_HOP - 1):
            cw[h].wait_recv()
            cw[h + 1].start()
            ccw[h].wait_recv()
            ccw[h + 1].start()
            gemm(h + 1)
            gemm(7 - h)
        cw[N_HOP - 1].wait_recv()
        ccw[N_HOP - 1].wait_recv()
        gemm(4)

        for rdma in cw + ccw:
            rdma.wait_send()

    return pl.pallas_call(
        body,
        out_shape=jax.ShapeDtypeStruct((N_DEV * m_per, n_per), jnp.float32),
        in_specs=[
            pl.BlockSpec(memory_space=pltpu.VMEM),
            pl.BlockSpec(memory_space=pltpu.VMEM),
        ],
        out_specs=pl.BlockSpec(memory_space=pltpu.VMEM),
        scratch_shapes=[
            pltpu.VMEM((N_DEV - 1, m_per, k), jnp.float32),
            pltpu.SemaphoreType.DMA((N_HOP,)),
            pltpu.SemaphoreType.DMA((N_HOP,)),
            pltpu.SemaphoreType.DMA((N_HOP,)),
            pltpu.SemaphoreType.DMA((N_HOP,)),
        ],
        compiler_params=pltpu.CompilerParams(collective_id=0),
    )(x, w_mat)


# device time: 68251 ns/iter; 2.6933x vs baseline; 1.3887x over previous
import jax
import jax.numpy as jnp
from jax import lax
from jax.experimental import pallas as pl
from jax.experimental.pallas import tpu as pltpu

N_DEV = 8

_AXES = ((1, 0, 0), (0, 1, 0), (0, 0, 1))


def _coords_of(p):
    q = p & 3
    g = q ^ (q >> 1)
    return g & 1, (g >> 1) & 1, p >> 2


def _idx_of(cx, cy, cz):
    return cz * 4 + cy * 2 + (cx ^ cy)


def _xor(c, m):
    return tuple(u ^ v for u, v in zip(c, m))


def _flip(c, m):
    return tuple(u ^ v if v else u for u, v in zip(c, m))


_MASKS = [
    (1, 0, 0), (0, 1, 0), (0, 0, 1),
    (1, 1, 0), (1, 0, 1), (0, 1, 1),
    (1, 1, 1),
]
_SLOT = {m: i for i, m in enumerate(_MASKS)}

_H2IDX = {}
for _a in range(3):
    for _e in range(3):
        if _e != _a:
            _H2IDX[(_a, _e)] = len(_H2IDX)

_THIRD = {0: (0, 88), 1: (88, 88), 2: (176, 80)}


def kernel(x, w_mat):
    m_per, k = x.shape
    _, n_per = w_mat.shape
    half = m_per // 2

    def body(x_ref, w_ref, out_ref, comm_ref, s1, r1, s2, r2, s3, r3):
        my = lax.axis_index("i")
        mc = _coords_of(my)
        nbr = [_idx_of(*_flip(mc, _AXES[a])) for a in range(3)]

        barrier_sem = pltpu.get_barrier_semaphore()
        for a in range(3):
            pl.semaphore_signal(
                barrier_sem, inc=1, device_id=(nbr[a],),
                device_id_type=pl.DeviceIdType.MESH,
            )
        pl.semaphore_wait(barrier_sem, 3)

        sends = []

        def copy(src, dst, ssem, rsem, dev):
            rd = pltpu.make_async_remote_copy(
                src_ref=src, dst_ref=dst, send_sem=ssem, recv_sem=rsem,
                device_id=(dev,), device_id_type=pl.DeviceIdType.MESH,
            )
            rd.start()
            sends.append(rd)
            return rd

        def gemm(slot):
            src = x_ref if slot is None else comm_ref.at[slot]
            og = my if slot is None else _idx_of(*_flip(mc, _MASKS[slot]))
            out_ref[pl.ds(og * m_per, m_per), :] = jnp.dot(
                src[...], w_ref[...], preferred_element_type=jnp.float32
            )

        h1 = [
            copy(x_ref, comm_ref.at[_SLOT[_AXES[a]]], s1.at[a], r1.at[a],
                 nbr[a])
            for a in range(3)
        ]
        gemm(None)

        h2 = {}
        for a in range(3):
            h1[a].wait_recv()
            sa = _SLOT[_AXES[a]]
            for e in range(3):
                if e == a:
                    continue
                lo = 0 if a < e else half
                ps = _SLOT[_xor(_AXES[a], _AXES[e])]
                idx = _H2IDX[(a, e)]
                h2[(a, e)] = copy(
                    comm_ref.at[sa, pl.ds(lo, half)],
                    comm_ref.at[ps, pl.ds(lo, half)],
                    s2.at[idx], r2.at[idx], nbr[e],
                )
            gemm(sa)

        h3 = {}
        for a, b in ((0, 1), (0, 2), (1, 2)):
            ps = _SLOT[_xor(_AXES[a], _AXES[b])]
            c = 3 - a - b
            h2[(a, b)].wait_recv()
            h2[(b, a)].wait_recv()
            lo, sz = _THIRD[c]
            h3[c] = copy(
                comm_ref.at[ps, pl.ds(lo, sz)],
                comm_ref.at[6, pl.ds(lo, sz)],
                s3.at[c], r3.at[c], nbr[c],
            )
            gemm(ps)

        for c in range(3):
            h3[c].wait_recv()
        gemm(6)

        for rd in sends:
            rd.wait_send()

    return pl.pallas_call(
        body,
        out_shape=jax.ShapeDtypeStruct((N_DEV * m_per, n_per), jnp.float32),
        in_specs=[
            pl.BlockSpec(memory_space=pltpu.VMEM),
            pl.BlockSpec(memory_space=pltpu.VMEM),
        ],
        out_specs=pl.BlockSpec(memory_space=pltpu.VMEM),
        scratch_shapes=[
            pltpu.VMEM((7, m_per, k), jnp.float32),
            pltpu.SemaphoreType.DMA((3,)),
            pltpu.SemaphoreType.DMA((3,)),
            pltpu.SemaphoreType.DMA((6,)),
            pltpu.SemaphoreType.DMA((6,)),
            pltpu.SemaphoreType.DMA((3,)),
            pltpu.SemaphoreType.DMA((3,)),
        ],
        compiler_params=pltpu.CompilerParams(collective_id=0),
    )(x, w_mat)


# device time: 66762 ns/iter; 2.7534x vs baseline; 1.0223x over previous
import jax
import jax.numpy as jnp
from jax import lax
from jax.experimental import pallas as pl
from jax.experimental.pallas import tpu as pltpu

N_DEV = 8

_AXES = ((1, 0, 0), (0, 1, 0), (0, 0, 1))


def _coords_of(p):
    q = p & 3
    g = q ^ (q >> 1)
    return g & 1, (g >> 1) & 1, p >> 2


def _idx_of(cx, cy, cz):
    return cz * 4 + cy * 2 + (cx ^ cy)


def _xor(c, m):
    return tuple(u ^ v for u, v in zip(c, m))


def _flip(c, m):
    return tuple(u ^ v if v else u for u, v in zip(c, m))


_MASKS = [
    (1, 0, 0), (0, 1, 0), (0, 0, 1),
    (1, 1, 0), (1, 0, 1), (0, 1, 1),
    (1, 1, 1),
]
_SLOT = {m: i for i, m in enumerate(_MASKS)}

_H2IDX = {}
for _a in range(3):
    for _e in range(3):
        if _e != _a:
            _H2IDX[(_a, _e)] = len(_H2IDX)

_THIRD = {0: (0, 88), 1: (88, 88), 2: (176, 80)}


def kernel(x, w_mat):
    m_per, k = x.shape
    _, n_per = w_mat.shape
    half = m_per // 2

    def body(x_ref, w_ref, out_ref, comm_ref, s1, r1, s2, r2, s3, r3):
        my = lax.axis_index("i")
        mc = _coords_of(my)
        nbr = [_idx_of(*_flip(mc, _AXES[a])) for a in range(3)]

        barrier_sem = pltpu.get_barrier_semaphore()
        for a in range(3):
            pl.semaphore_signal(
                barrier_sem, inc=1, device_id=(nbr[a],),
                device_id_type=pl.DeviceIdType.MESH,
            )
        pl.semaphore_wait(barrier_sem, 3)

        sends = []

        def copy(src, dst, ssem, rsem, dev):
            rd = pltpu.make_async_remote_copy(
                src_ref=src, dst_ref=dst, send_sem=ssem, recv_sem=rsem,
                device_id=(dev,), device_id_type=pl.DeviceIdType.MESH,
            )
            rd.start()
            sends.append(rd)
            return rd

        def gemm(slot):
            src = x_ref if slot is None else comm_ref.at[slot]
            og = my if slot is None else _idx_of(*_flip(mc, _MASKS[slot]))
            out_ref[pl.ds(og * m_per, m_per), :] = jnp.dot(
                src[...], w_ref[...], preferred_element_type=jnp.float32
            )

        h1 = [
            copy(x_ref, comm_ref.at[_SLOT[_AXES[a]]], s1.at[a], r1.at[a],
                 nbr[a])
            for a in range(3)
        ]
        gemm(None)

        h2 = {}
        for a in range(3):
            h1[a].wait_recv()
            sa = _SLOT[_AXES[a]]
            for e in range(3):
                if e == a:
                    continue
                lo = 0 if a < e else half
                ps = _SLOT[_xor(_AXES[a], _AXES[e])]
                idx = _H2IDX[(a, e)]
                h2[(a, e)] = copy(
                    comm_ref.at[sa, pl.ds(lo, half)],
                    comm_ref.at[ps, pl.ds(lo, half)],
                    s2.at[idx], r2.at[idx], nbr[e],
                )
        for a in range(3):
            gemm(_SLOT[_AXES[a]])

        h3 = {}
        for a, b in ((0, 1), (0, 2), (1, 2)):
            ps = _SLOT[_xor(_AXES[a], _AXES[b])]
            c = 3 - a - b
            h2[(a, b)].wait_recv()
            h2[(b, a)].wait_recv()
            lo, sz = _THIRD[c]
            h3[c] = copy(
                comm_ref.at[ps, pl.ds(lo, sz)],
                comm_ref.at[6, pl.ds(lo, sz)],
                s3.at[c], r3.at[c], nbr[c],
            )
        for a, b in ((0, 1), (0, 2), (1, 2)):
            gemm(_SLOT[_xor(_AXES[a], _AXES[b])])

        for c in range(3):
            h3[c].wait_recv()
        gemm(6)

        for rd in sends:
            rd.wait_send()

    return pl.pallas_call(
        body,
        out_shape=jax.ShapeDtypeStruct((N_DEV * m_per, n_per), jnp.float32),
        in_specs=[
            pl.BlockSpec(memory_space=pltpu.VMEM),
            pl.BlockSpec(memory_space=pltpu.VMEM),
        ],
        out_specs=pl.BlockSpec(memory_space=pltpu.VMEM),
        scratch_shapes=[
            pltpu.VMEM((7, m_per, k), jnp.float32),
            pltpu.SemaphoreType.DMA((3,)),
            pltpu.SemaphoreType.DMA((3,)),
            pltpu.SemaphoreType.DMA((6,)),
            pltpu.SemaphoreType.DMA((6,)),
            pltpu.SemaphoreType.DMA((3,)),
            pltpu.SemaphoreType.DMA((3,)),
        ],
        compiler_params=pltpu.CompilerParams(collective_id=0),
    )(x, w_mat)
